# baseline (speedup 1.0000x reference)
import jax
import jax.numpy as jnp
from jax import lax
from jax.experimental import pallas as pl
from jax.experimental.pallas import tpu as pltpu
from jax.experimental.pallas import tpu_sc as plsc

_NC = 2
_NS = 16
_NW = _NC * _NS
_L = 16

_CHUNK = 128


def _make_lookup(batch, fields, feat, num_emb):
    assert batch % _NW == 0
    b_per_w = batch // _NW
    nq = b_per_w // _CHUNK
    assert b_per_w % _CHUNK == 0
    nu = fields * nq
    assert nu % 2 == 0

    mesh = plsc.VectorSubcoreMesh(
        core_axis_name="c", subcore_axis_name="s",
        num_cores=_NC, num_subcores=_NS)

    @pl.kernel(
        out_type=jax.ShapeDtypeStruct((fields, feat, batch), jnp.float32),
        mesh=mesh,
        compiler_params=pltpu.CompilerParams(
            use_tc_tiling_on_sc=False, needs_layout_passes=False),
        scratch_types=[
            pltpu.VMEM((_CHUNK,), jnp.int32),
            pltpu.VMEM((_CHUNK,), jnp.int32),
            pltpu.VMEM((_CHUNK, 16), jnp.float32),
            pltpu.VMEM((_CHUNK, 16), jnp.float32),
            pltpu.VMEM((feat, b_per_w), jnp.float32),
            pltpu.SemaphoreType.DMA,
            pltpu.SemaphoreType.DMA,
        ],
    )
    def k(table_hbm, idx_hbm, out_hbm, idx0, idx1, g0, g1, o_tile,
          sem0, sem1):
        wid = lax.axis_index("s") * _NC + lax.axis_index("c")
        b0 = wid * b_per_w
        iota = lax.iota(jnp.int32, _L)
        cols = [jnp.full((_L,), j, jnp.int32) for j in range(feat)]

        idxb = (idx0, idx1)
        gbuf = (g0, g1)
        sems = (sem0, sem1)

        def stage(u, par):
            f = u // nq
            q = u % nq
            src = f * batch + b0 + q * _CHUNK
            pltpu.sync_copy(idx_hbm.at[pl.ds(src, _CHUNK)], idxb[par])
            pltpu.async_copy(table_hbm.at[idxb[par]], gbuf[par], sems[par])

        def gwait(par):
            pltpu.make_async_copy(
                table_hbm.at[idxb[par]], gbuf[par], sems[par]).wait()

        def extract(u, par):
            q = u % nq
            col0 = q * _CHUNK
            g = gbuf[par]
            for t in range(_CHUNK // _L):
                rows = iota + t * _L
                for j in range(feat):
                    vals = plsc.load_gather(g, [rows, cols[j]])
                    o_tile[j, pl.ds(col0 + t * _L, _L)] = vals

        def flush(u):
            f = u // nq
            pltpu.sync_copy(o_tile, out_hbm.at[f, :, pl.ds(b0, b_per_w)])

        stage(0, 0)

        def body(s, carry):
            u = s * 2
            stage(u + 1, 1)
            gwait(0)
            extract(u, 0)

            @pl.when((u % nq) == (nq - 1))
            def _():
                flush(u)

            @pl.when(s < (nu // 2 - 1))
            def _():
                stage(u + 2, 0)
            gwait(1)
            extract(u + 1, 1)

            @pl.when(((u + 1) % nq) == (nq - 1))
            def _():
                flush(u + 1)

            return carry

        lax.fori_loop(0, nu // 2, body, 0)

    return k


def kernel(inputs, embedding):
    batch, fields = inputs.shape
    num_emb, feat = embedding.shape
    idx_fm = jnp.transpose(inputs).reshape(batch * fields).astype(jnp.int32)
    call = _make_lookup(batch, fields, feat, num_emb)
    out_t = call(embedding, idx_fm)
    return jnp.transpose(out_t, (2, 0, 1))


# speedup vs baseline: 1.0587x; 1.0587x over previous
import jax
import jax.numpy as jnp
from jax import lax
from jax.experimental import pallas as pl
from jax.experimental.pallas import tpu as pltpu
from jax.experimental.pallas import tpu_sc as plsc

_NC = 2
_NS = 16
_NW = _NC * _NS
_L = 16

_CHUNK = 128
_NBUF = 4


def _make_lookup(batch, fields, feat, num_emb):
    assert batch % _NW == 0
    b_per_w = batch // _NW
    nq = b_per_w // _CHUNK
    assert nq == _NBUF and fields % 2 == 0
    nu = fields * nq

    mesh = plsc.VectorSubcoreMesh(
        core_axis_name="c", subcore_axis_name="s",
        num_cores=_NC, num_subcores=_NS)

    @pl.kernel(
        out_type=jax.ShapeDtypeStruct((fields, feat, batch), jnp.float32),
        mesh=mesh,
        compiler_params=pltpu.CompilerParams(
            use_tc_tiling_on_sc=False, needs_layout_passes=False),
        scratch_types=[
            pltpu.VMEM((fields * b_per_w,), jnp.int32),
            pltpu.VMEM((_CHUNK, 16), jnp.float32),
            pltpu.VMEM((_CHUNK, 16), jnp.float32),
            pltpu.VMEM((_CHUNK, 16), jnp.float32),
            pltpu.VMEM((_CHUNK, 16), jnp.float32),
            pltpu.VMEM((feat, b_per_w), jnp.float32),
            pltpu.VMEM((feat, b_per_w), jnp.float32),
            pltpu.SemaphoreType.DMA,
            pltpu.SemaphoreType.DMA,
            pltpu.SemaphoreType.DMA,
            pltpu.SemaphoreType.DMA,
            pltpu.SemaphoreType.DMA,
            pltpu.SemaphoreType.DMA,
            pltpu.SemaphoreType.DMA,
        ],
    )
    def k(table_hbm, idx_hbm, out_hbm, idx_all, g0, g1, g2, g3, ot0, ot1,
          isem, gsem0, gsem1, gsem2, gsem3, osem0, osem1):
        wid = lax.axis_index("s") * _NC + lax.axis_index("c")
        b0 = wid * b_per_w
        iota = lax.iota(jnp.int32, _L)
        cols = [jnp.full((_L,), j, jnp.int32) for j in range(feat)]

        gbuf = (g0, g1, g2, g3)
        gsems = (gsem0, gsem1, gsem2, gsem3)
        otile = (ot0, ot1)
        osems = (osem0, osem1)

        icps = [
            pltpu.async_copy(
                idx_hbm.at[pl.ds(f * batch + b0, b_per_w)],
                idx_all.at[pl.ds(f * b_per_w, b_per_w)], isem)
            for f in range(fields)
        ]
        for cp in icps:
            cp.wait()

        def ichunk(u):
            return idx_all.at[pl.ds(u * _CHUNK, _CHUNK)]

        def fire(u, par):
            pltpu.async_copy(table_hbm.at[ichunk(u)], gbuf[par], gsems[par])

        def gwait(u, par):
            pltpu.make_async_copy(
                table_hbm.at[ichunk(u)], gbuf[par], gsems[par]).wait()

        def extract(par, o_ref, col0):
            g = gbuf[par]
            for t in range(_CHUNK // _L):
                rows = iota + t * _L
                for j in range(feat):
                    o_ref[j, pl.ds(col0 + t * _L, _L)] = (
                        plsc.load_gather(g, [rows, cols[j]]))

        def oslice(f):
            return out_hbm.at[f, :, pl.ds(b0, b_per_w)]

        for u in range(_NBUF - 1):
            fire(u, u)

        def body(s, carry):
            u0 = s * 2 * nq
            fa = s * 2
            fb = s * 2 + 1

            @pl.when(s > 0)
            def _():
                pltpu.make_async_copy(otile[0], oslice(fa), osems[0]).wait()
                pltpu.make_async_copy(otile[1], oslice(fb), osems[1]).wait()

            for p in range(2 * nq):
                u = u0 + p
                par = p % _NBUF

                @pl.when(u + _NBUF - 1 < nu)
                def _():
                    fire(u + _NBUF - 1, (p + _NBUF - 1) % _NBUF)
                gwait(u, par)
                extract(par, otile[p // nq], (p % nq) * _CHUNK)

            pltpu.async_copy(otile[0], oslice(fa), osems[0])
            pltpu.async_copy(otile[1], oslice(fb), osems[1])
            return carry

        lax.fori_loop(0, fields // 2, body, 0)
        pltpu.make_async_copy(otile[0], oslice(fields - 2), osems[0]).wait()
        pltpu.make_async_copy(otile[1], oslice(fields - 1), osems[1]).wait()

    return k


def kernel(inputs, embedding):
    batch, fields = inputs.shape
    num_emb, feat = embedding.shape
    idx_fm = jnp.transpose(inputs).reshape(batch * fields).astype(jnp.int32)
    call = _make_lookup(batch, fields, feat, num_emb)
    out_t = call(embedding, idx_fm)
    return jnp.transpose(out_t, (2, 0, 1))
